# baseline (speedup 1.0000x reference)
import functools
import math

import jax
import jax.numpy as jnp
from jax import lax
from jax.experimental import pallas as pl
from jax.experimental.pallas import tpu as pltpu
from jax.experimental.pallas import tpu_sc as plsc

N = 10000
E = 160000
NDIM = 256
H = 8
DK = NDIM // H
DFF = 4 * NDIM

NC = 2
NS = 16
HD = NDIM // 2
HH = H // 2
UW = 144

ROW_BLK = 1000

CE = 48
NG = CE // 16
EPT = E // NS
NCH = EPT // CE
NPAIR = NCH // 2
CT = EPT - NCH * CE
RPT = N // NS
ISQ = 1.0 / math.sqrt(DK)

_mesh = plsc.VectorSubcoreMesh(
    core_axis_name="c", subcore_axis_name="s", num_cores=NC, num_subcores=NS)
_sc_params = pltpu.CompilerParams(use_tc_tiling_on_sc=False,
                                  needs_layout_passes=False)


def _qkv_body(x_ref, w_ref, bq_ref, q_ref, k_ref, v_ref):
    acc = jnp.dot(x_ref[...], w_ref[...], preferred_element_type=jnp.float32)
    q_ref[...] = acc[:, :NDIM] + bq_ref[...]
    k_ref[...] = acc[:, NDIM:2 * NDIM]
    v_ref[...] = acc[:, 2 * NDIM:]


def _qkv(x, wqkv, bq):
    out = jax.ShapeDtypeStruct((N, NDIM), jnp.float32)
    return pl.pallas_call(
        _qkv_body,
        grid=(N // ROW_BLK,),
        in_specs=[
            pl.BlockSpec((ROW_BLK, NDIM), lambda i: (i, 0)),
            pl.BlockSpec((NDIM, 3 * NDIM), lambda i: (0, 0)),
            pl.BlockSpec((1, NDIM), lambda i: (0, 0)),
        ],
        out_specs=[pl.BlockSpec((ROW_BLK, NDIM), lambda i: (i, 0))] * 3,
        out_shape=[out, out, out],
    )(x, wqkv, bq.reshape(1, NDIM))


def _edge_body(k2_hbm, q2_hbm, v2_hbm, ei2_hbm, zeros_hbm, out_hbm,
               idx2A, idx2B, sidxA, sidxB, gdidxA, gdidxB, didxA, didxB,
               didxS, kA, kB, qA, qB, vbuf, ubuf, sbufS, cvtab,
               idx2T, sidxT, gdidxT, didxT,
               accum, semA, semB, semV, semS, semIA, semIB):
    c = lax.axis_index("c")
    s = lax.axis_index("s")
    ebase = s * EPT
    rbase = s * RPT

    pltpu.sync_copy(zeros_hbm, accum.at[pl.ds(rbase, RPT)])
    pltpu.sync_copy(zeros_hbm.at[pl.ds(0, CE)], ubuf)

    def fire_idx(x, idx2, semI):
        pltpu.async_copy(ei2_hbm.at[pl.ds(ebase + x * CE, CE)], idx2, semI)

    def idx_transform(idx2, sidx, gdidx, didx, n):
        for g in range(n // 16):
            ev2 = (lax.iota(jnp.int32, 16) + g * 16) * 2
            sv = plsc.load_gather(idx2, [ev2 // 2, jnp.zeros((16,), jnp.int32)])
            dv = plsc.load_gather(idx2, [ev2 // 2, jnp.ones((16,), jnp.int32)])
            sl = pl.ds(g * 16, 16)
            sidx[sl] = sv + sv + c
            gdidx[sl] = dv + dv + c
            didx[sl] = dv

    def fire_kq(x, sidx, gdidx, kb, qb, sem):
        pltpu.async_copy(k2_hbm.at[sidx], kb, sem)
        pltpu.async_copy(q2_hbm.at[gdidx], qb, sem)

    def prep(x, idx2, semI, sidx, gdidx, didx, kb, qb, sem):
        pltpu.make_async_copy(ei2_hbm.at[pl.ds(0, CE)], idx2, semI).wait()
        idx_transform(idx2, sidx, gdidx, didx, CE)
        fire_kq(x, sidx, gdidx, kb, qb, sem)

        @pl.when(x + 2 < NCH)
        def _():
            fire_idx(x + 2, idx2, semI)

    def drain_kq(kb, qb, sem):
        pltpu.make_async_copy(k2_hbm.at[pl.ds(0, CE)], kb, sem).wait()
        pltpu.make_async_copy(k2_hbm.at[pl.ds(0, CE)], qb, sem).wait()

    _lane = lax.iota(jnp.int32, 16)

    def cvinit(ci, carry):
        cvtab[pl.ds(ci * 16, 16)] = (_lane + ci) & (DK - 1)
        return carry

    lax.fori_loop(0, DK, cvinit, 0)

    def score_loop(ngroups, kb, qb):
        def gbody(g, carry):
            ev = _lane + g * 16
            for h in range(HH):
                acc = jnp.zeros((16,), jnp.float32)
                cvb = cvtab[pl.ds(0, 16)]
                for i in range(DK):
                    cv = cvb + h * DK
                    kc = plsc.load_gather(kb, [ev, cv])
                    qc = plsc.load_gather(qb, [ev, cv])
                    acc = acc + kc * qc
                    if i < DK - 1:
                        cvb = (cvb + 1) & (DK - 1)
                sh = jnp.exp(jnp.clip(acc * ISQ, -5.0, 5.0))
                sbufS[h, pl.ds(g * 16, 16)] = sh
            return carry

        lax.fori_loop(0, ngroups, gbody, 0)

    def weight_loop(ngroups):
        def gbody(g, carry):
            ev = _lane + g * 16
            sl = pl.ds(g * 16, 16)
            for h in range(HH):
                sv = sbufS[h, sl]
                plsc.store_scatter(
                    ubuf, [ev, jnp.full((16,), HD + h, jnp.int32)], sv)

                cvb = cvtab[pl.ds(0, 16)]
                for i in range(DK):
                    cv = cvb + h * DK
                    vc = plsc.load_gather(vbuf, [ev, cv])
                    plsc.store_scatter(ubuf, [ev, cv], vc * sv)
                    if i < DK - 1:
                        cvb = (cvb + 1) & (DK - 1)
            return carry

        lax.fori_loop(0, ngroups, gbody, 0)

    def fire_scatter(didx, nrows):
        for g in range(nrows // 16):
            sl = pl.ds(g * 16, 16)
            didxS[sl] = didx[sl]
        pltpu.async_copy(ubuf.at[pl.ds(0, nrows)],
                         accum.at[didxS.at[pl.ds(0, nrows)]], semS, add=True)

    def drain_scatter(nrows):
        pltpu.make_async_copy(ubuf.at[pl.ds(0, nrows)],
                              accum.at[pl.ds(0, nrows)], semS).wait()

    plsc.subcore_barrier()

    fire_idx(0, idx2A, semIA)
    fire_idx(1, idx2B, semIB)
    prep(0, idx2A, semIA, sidxA, gdidxA, didxA, kA, qA, semA)

    def pair(j, carry):
        a = 2 * j
        prep(a + 1, idx2B, semIB, sidxB, gdidxB, didxB, kB, qB, semB)
        drain_kq(kA, qA, semA)
        pltpu.async_copy(v2_hbm.at[sidxA], vbuf, semV)
        score_loop(NG, kA, qA)
        pltpu.make_async_copy(k2_hbm.at[pl.ds(0, CE)], vbuf, semV).wait()

        @pl.when(j > 0)
        def _():
            drain_scatter(CE)

        weight_loop(NG)
        fire_scatter(didxA, CE)

        @pl.when(j < NPAIR - 1)
        def _():
            prep(a + 2, idx2A, semIA, sidxA, gdidxA, didxA, kA, qA, semA)

        drain_kq(kB, qB, semB)
        pltpu.async_copy(v2_hbm.at[sidxB], vbuf, semV)
        score_loop(NG, kB, qB)
        pltpu.make_async_copy(k2_hbm.at[pl.ds(0, CE)], vbuf, semV).wait()
        drain_scatter(CE)
        weight_loop(NG)
        fire_scatter(didxB, CE)
        return carry

    lax.fori_loop(0, NPAIR, pair, 0)
    drain_scatter(CE)

    pltpu.sync_copy(ei2_hbm.at[pl.ds(ebase + NCH * CE, CT)], idx2T)
    idx_transform(idx2T, sidxT, gdidxT, didxT, CT)
    pltpu.sync_copy(k2_hbm.at[sidxT], kA.at[pl.ds(0, CT)])
    pltpu.sync_copy(q2_hbm.at[gdidxT], qA.at[pl.ds(0, CT)])
    pltpu.sync_copy(v2_hbm.at[sidxT], vbuf.at[pl.ds(0, CT)])
    score_loop(CT // 16, kA, qA)
    weight_loop(CT // 16)
    pltpu.sync_copy(ubuf.at[pl.ds(0, CT)], accum.at[didxT], add=True)

    plsc.subcore_barrier()

    @pl.when(c == 0)
    def _():
        pltpu.sync_copy(accum.at[pl.ds(rbase, RPT)],
                        out_hbm.at[0, pl.ds(rbase, RPT)])

    @pl.when(c == 1)
    def _():
        pltpu.sync_copy(accum.at[pl.ds(rbase, RPT)],
                        out_hbm.at[1, pl.ds(rbase, RPT)])


def _edge(k2, q2, v2, ei2, zeros):
    f = pl.kernel(
        _edge_body,
        out_type=jax.ShapeDtypeStruct((2, N, UW), jnp.float32),
        mesh=_mesh,
        scratch_types=[
            pltpu.VMEM((CE, 2), jnp.int32),
            pltpu.VMEM((CE, 2), jnp.int32),
            pltpu.VMEM((CE,), jnp.int32),
            pltpu.VMEM((CE,), jnp.int32),
            pltpu.VMEM((CE,), jnp.int32),
            pltpu.VMEM((CE,), jnp.int32),
            pltpu.VMEM((CE,), jnp.int32),
            pltpu.VMEM((CE,), jnp.int32),
            pltpu.VMEM((CE,), jnp.int32),
            pltpu.VMEM((CE, HD), jnp.float32),
            pltpu.VMEM((CE, HD), jnp.float32),
            pltpu.VMEM((CE, HD), jnp.float32),
            pltpu.VMEM((CE, HD), jnp.float32),
            pltpu.VMEM((CE, HD), jnp.float32),
            pltpu.VMEM((CE, UW), jnp.float32),
            pltpu.VMEM((HH, CE), jnp.float32),
            pltpu.VMEM((DK * 16,), jnp.int32),
            pltpu.VMEM((CT, 2), jnp.int32),
            pltpu.VMEM((CT,), jnp.int32),
            pltpu.VMEM((CT,), jnp.int32),
            pltpu.VMEM((CT,), jnp.int32),
            pltpu.VMEM_SHARED((N, UW), jnp.float32),
            pltpu.SemaphoreType.DMA,
            pltpu.SemaphoreType.DMA,
            pltpu.SemaphoreType.DMA,
            pltpu.SemaphoreType.DMA,
            pltpu.SemaphoreType.DMA,
            pltpu.SemaphoreType.DMA,
        ],
        compiler_params=_sc_params,
    )
    return f(k2, q2, v2, ei2, zeros)


def _ln(h, g, b, eps=1e-5):
    m = jnp.mean(h, axis=-1, keepdims=True)
    cc = h - m
    v = jnp.mean(cc * cc, axis=-1, keepdims=True)
    return cc * lax.rsqrt(v + eps) * g + b


def _post_body(x_ref, wv_ref, zb_ref, wo_ref, bo_ref, lng_ref, lnb_ref,
               w1_ref, b1_ref, w2_ref, b2_ref, ln2g_ref, ln2b_ref, out_ref):
    o = wv_ref[...] / (zb_ref[...] + 1e-12)
    x = x_ref[...]
    h = _ln(x + jnp.dot(o, wo_ref[...], preferred_element_type=jnp.float32)
            + bo_ref[...], lng_ref[...], lnb_ref[...])
    f = jnp.maximum(jnp.dot(h, w1_ref[...], preferred_element_type=jnp.float32)
                    + b1_ref[...], 0.0)
    out_ref[...] = _ln(h + jnp.dot(f, w2_ref[...], preferred_element_type=jnp.float32)
                       + b2_ref[...], ln2g_ref[...], ln2b_ref[...])


def _post(x, wv, zb, Wo, bo, ln_g, ln_b, W1, b1, W2, b2, ln2_g, ln2_b):
    row = lambda i: (i, 0)
    fixed = lambda i: (0, 0)
    return pl.pallas_call(
        _post_body,
        grid=(N // ROW_BLK,),
        in_specs=[
            pl.BlockSpec((ROW_BLK, NDIM), row),
            pl.BlockSpec((ROW_BLK, NDIM), row),
            pl.BlockSpec((ROW_BLK, NDIM), row),
            pl.BlockSpec((NDIM, NDIM), fixed),
            pl.BlockSpec((1, NDIM), fixed),
            pl.BlockSpec((1, NDIM), fixed),
            pl.BlockSpec((1, NDIM), fixed),
            pl.BlockSpec((NDIM, DFF), fixed),
            pl.BlockSpec((1, DFF), fixed),
            pl.BlockSpec((DFF, NDIM), fixed),
            pl.BlockSpec((1, NDIM), fixed),
            pl.BlockSpec((1, NDIM), fixed),
            pl.BlockSpec((1, NDIM), fixed),
        ],
        out_specs=pl.BlockSpec((ROW_BLK, NDIM), row),
        out_shape=jax.ShapeDtypeStruct((N, NDIM), jnp.float32),
    )(x, wv, zb, Wo, bo.reshape(1, NDIM), ln_g.reshape(1, NDIM),
      ln_b.reshape(1, NDIM), W1, b1.reshape(1, DFF), W2, b2.reshape(1, NDIM),
      ln2_g.reshape(1, NDIM), ln2_b.reshape(1, NDIM))


def kernel(x, edge_index, Wq, bq, Wk, Wv, Wo, bo, ln_g, ln_b, W1, b1, W2, b2,
           ln2_g, ln2_b):
    wqkv = jnp.concatenate([Wq, Wk, Wv], axis=1)
    q, k, v = _qkv(x, wqkv, bq)

    ei2 = edge_index.T
    zeros = jnp.zeros((RPT, UW), jnp.float32)
    agg = _edge(k.reshape(2 * N, HD), q.reshape(2 * N, HD),
                v.reshape(2 * N, HD), ei2, zeros)

    wv = jnp.concatenate([agg[0, :, :HD], agg[1, :, :HD]], axis=1)
    z = jnp.concatenate([agg[0, :, HD:HD + HH], agg[1, :, HD:HD + HH]], axis=1)
    zb = jnp.broadcast_to(z[:, :, None], (N, H, DK)).reshape(N, NDIM)
    return _post(x, wv, zb, Wo, bo, ln_g, ln_b, W1, b1, W2, b2, ln2_g, ln2_b)


# speedup vs baseline: 1.1113x; 1.1113x over previous
import functools
import math

import jax
import jax.numpy as jnp
from jax import lax
from jax.experimental import pallas as pl
from jax.experimental.pallas import tpu as pltpu
from jax.experimental.pallas import tpu_sc as plsc

N = 10000
E = 160000
NDIM = 256
H = 8
DK = NDIM // H
DFF = 4 * NDIM

NC = 2
NS = 16
HD = NDIM // 2
HH = H // 2
UW = 144

ROW_BLK = 1000

CE = 48
NG = CE // 16
EPT = E // NS
NCH = EPT // CE
NPAIR = NCH // 2
CT = EPT - NCH * CE
RPT = N // NS
ISQ = 1.0 / math.sqrt(DK)

_mesh = plsc.VectorSubcoreMesh(
    core_axis_name="c", subcore_axis_name="s", num_cores=NC, num_subcores=NS)
_sc_params = pltpu.CompilerParams(use_tc_tiling_on_sc=False,
                                  needs_layout_passes=False)


def _qkv_body(x_ref, w_ref, bq_ref, q_ref, k_ref, v_ref):
    acc = jnp.dot(x_ref[...], w_ref[...], preferred_element_type=jnp.float32)
    q_ref[...] = acc[:, :NDIM] + bq_ref[...]
    k_ref[...] = acc[:, NDIM:2 * NDIM]
    v_ref[...] = acc[:, 2 * NDIM:]


def _qkv(x, wqkv, bq):
    out = jax.ShapeDtypeStruct((N, NDIM), jnp.float32)
    return pl.pallas_call(
        _qkv_body,
        grid=(N // ROW_BLK,),
        in_specs=[
            pl.BlockSpec((ROW_BLK, NDIM), lambda i: (i, 0)),
            pl.BlockSpec((NDIM, 3 * NDIM), lambda i: (0, 0)),
            pl.BlockSpec((1, NDIM), lambda i: (0, 0)),
        ],
        out_specs=[pl.BlockSpec((ROW_BLK, NDIM), lambda i: (i, 0))] * 3,
        out_shape=[out, out, out],
    )(x, wqkv, bq.reshape(1, NDIM))


def _edge_body(k2_hbm, q2_hbm, v2_hbm, ei2_hbm, zeros_hbm, out_hbm,
               idx2A, idx2B, sidxA, sidxB, gdidxA, gdidxB, didxA, didxB,
               didxS, kA, kB, qA, qB, vbuf, ubuf, sbufS, cvtab,
               idx2T, sidxT, gdidxT, didxT,
               accum, semA, semB, semV, semS, semIA, semIB):
    c = lax.axis_index("c")
    s = lax.axis_index("s")
    ebase = s * EPT
    rbase = s * RPT

    pltpu.sync_copy(zeros_hbm, accum.at[pl.ds(rbase, RPT)])
    pltpu.sync_copy(zeros_hbm.at[pl.ds(0, CE)], ubuf)

    def fire_idx(x, idx2, semI):
        pltpu.async_copy(ei2_hbm.at[pl.ds(ebase + x * CE, CE)], idx2, semI)

    def idx_transform(idx2, sidx, gdidx, didx, n):
        for g in range(n // 16):
            ev2 = (lax.iota(jnp.int32, 16) + g * 16) * 2
            sv = plsc.load_gather(idx2, [ev2 // 2, jnp.zeros((16,), jnp.int32)])
            dv = plsc.load_gather(idx2, [ev2 // 2, jnp.ones((16,), jnp.int32)])
            sl = pl.ds(g * 16, 16)
            sidx[sl] = sv + sv + c
            gdidx[sl] = dv + dv + c
            didx[sl] = dv

    def fire_kq(x, sidx, gdidx, kb, qb, sem):
        pltpu.async_copy(k2_hbm.at[sidx], kb, sem)
        pltpu.async_copy(q2_hbm.at[gdidx], qb, sem)

    def prep(x, idx2, semI, sidx, gdidx, didx, kb, qb, sem):
        pltpu.make_async_copy(ei2_hbm.at[pl.ds(0, CE)], idx2, semI).wait()
        idx_transform(idx2, sidx, gdidx, didx, CE)
        fire_kq(x, sidx, gdidx, kb, qb, sem)

        @pl.when(x + 2 < NCH)
        def _():
            fire_idx(x + 2, idx2, semI)

    def drain_kq(kb, qb, sem):
        pltpu.make_async_copy(k2_hbm.at[pl.ds(0, CE)], kb, sem).wait()
        pltpu.make_async_copy(k2_hbm.at[pl.ds(0, CE)], qb, sem).wait()

    _lane = lax.iota(jnp.int32, 16)

    def cvinit(ci, carry):
        cvtab[pl.ds(ci * 16, 16)] = (_lane + ci) & (DK - 1)
        return carry

    lax.fori_loop(0, DK, cvinit, 0)

    def score_loop(ngroups, kb, qb):
        def gbody(g, carry):
            ev = _lane + g * 16
            for h in range(HH):
                def cblk(b, acc):
                    cvb = cvtab[pl.ds(b * 256, 16)]
                    for i in range(16):
                        cv = cvb + h * DK
                        kc = plsc.load_gather(kb, [ev, cv])
                        qc = plsc.load_gather(qb, [ev, cv])
                        acc = acc + kc * qc
                        if i < 15:
                            cvb = (cvb + 1) & (DK - 1)
                    return acc

                acc = lax.fori_loop(0, DK // 16, cblk,
                                    jnp.zeros((16,), jnp.float32))
                sh = jnp.exp(jnp.clip(acc * ISQ, -5.0, 5.0))
                sbufS[h, pl.ds(g * 16, 16)] = sh
            return carry

        lax.fori_loop(0, ngroups, gbody, 0)

    def weight_loop(ngroups):
        def gbody(g, carry):
            ev = _lane + g * 16
            sl = pl.ds(g * 16, 16)
            for h in range(HH):
                sv = sbufS[h, sl]
                plsc.store_scatter(
                    ubuf, [ev, jnp.full((16,), HD + h, jnp.int32)], sv)

                def cblk(b, carry2):
                    cvb = cvtab[pl.ds(b * 256, 16)]
                    for i in range(16):
                        cv = cvb + h * DK
                        vc = plsc.load_gather(vbuf, [ev, cv])
                        plsc.store_scatter(ubuf, [ev, cv], vc * sv)
                        if i < 15:
                            cvb = (cvb + 1) & (DK - 1)
                    return carry2

                lax.fori_loop(0, DK // 16, cblk, 0)
            return carry

        lax.fori_loop(0, ngroups, gbody, 0)

    def fire_scatter(didx, nrows):
        for g in range(nrows // 16):
            sl = pl.ds(g * 16, 16)
            didxS[sl] = didx[sl]
        pltpu.async_copy(ubuf.at[pl.ds(0, nrows)],
                         accum.at[didxS.at[pl.ds(0, nrows)]], semS, add=True)

    def drain_scatter(nrows):
        pltpu.make_async_copy(ubuf.at[pl.ds(0, nrows)],
                              accum.at[pl.ds(0, nrows)], semS).wait()

    plsc.subcore_barrier()

    fire_idx(0, idx2A, semIA)
    fire_idx(1, idx2B, semIB)
    prep(0, idx2A, semIA, sidxA, gdidxA, didxA, kA, qA, semA)

    def pair(j, carry):
        a = 2 * j
        prep(a + 1, idx2B, semIB, sidxB, gdidxB, didxB, kB, qB, semB)
        drain_kq(kA, qA, semA)
        pltpu.async_copy(v2_hbm.at[sidxA], vbuf, semV)
        score_loop(NG, kA, qA)
        pltpu.make_async_copy(k2_hbm.at[pl.ds(0, CE)], vbuf, semV).wait()

        @pl.when(j > 0)
        def _():
            drain_scatter(CE)

        weight_loop(NG)
        fire_scatter(didxA, CE)

        @pl.when(j < NPAIR - 1)
        def _():
            prep(a + 2, idx2A, semIA, sidxA, gdidxA, didxA, kA, qA, semA)

        drain_kq(kB, qB, semB)
        pltpu.async_copy(v2_hbm.at[sidxB], vbuf, semV)
        score_loop(NG, kB, qB)
        pltpu.make_async_copy(k2_hbm.at[pl.ds(0, CE)], vbuf, semV).wait()
        drain_scatter(CE)
        weight_loop(NG)
        fire_scatter(didxB, CE)
        return carry

    lax.fori_loop(0, NPAIR, pair, 0)
    drain_scatter(CE)

    pltpu.sync_copy(ei2_hbm.at[pl.ds(ebase + NCH * CE, CT)], idx2T)
    idx_transform(idx2T, sidxT, gdidxT, didxT, CT)
    pltpu.sync_copy(k2_hbm.at[sidxT], kA.at[pl.ds(0, CT)])
    pltpu.sync_copy(q2_hbm.at[gdidxT], qA.at[pl.ds(0, CT)])
    pltpu.sync_copy(v2_hbm.at[sidxT], vbuf.at[pl.ds(0, CT)])
    score_loop(CT // 16, kA, qA)
    weight_loop(CT // 16)
    pltpu.sync_copy(ubuf.at[pl.ds(0, CT)], accum.at[didxT], add=True)

    plsc.subcore_barrier()

    @pl.when(c == 0)
    def _():
        pltpu.sync_copy(accum.at[pl.ds(rbase, RPT)],
                        out_hbm.at[0, pl.ds(rbase, RPT)])

    @pl.when(c == 1)
    def _():
        pltpu.sync_copy(accum.at[pl.ds(rbase, RPT)],
                        out_hbm.at[1, pl.ds(rbase, RPT)])


def _edge(k2, q2, v2, ei2, zeros):
    f = pl.kernel(
        _edge_body,
        out_type=jax.ShapeDtypeStruct((2, N, UW), jnp.float32),
        mesh=_mesh,
        scratch_types=[
            pltpu.VMEM((CE, 2), jnp.int32),
            pltpu.VMEM((CE, 2), jnp.int32),
            pltpu.VMEM((CE,), jnp.int32),
            pltpu.VMEM((CE,), jnp.int32),
            pltpu.VMEM((CE,), jnp.int32),
            pltpu.VMEM((CE,), jnp.int32),
            pltpu.VMEM((CE,), jnp.int32),
            pltpu.VMEM((CE,), jnp.int32),
            pltpu.VMEM((CE,), jnp.int32),
            pltpu.VMEM((CE, HD), jnp.float32),
            pltpu.VMEM((CE, HD), jnp.float32),
            pltpu.VMEM((CE, HD), jnp.float32),
            pltpu.VMEM((CE, HD), jnp.float32),
            pltpu.VMEM((CE, HD), jnp.float32),
            pltpu.VMEM((CE, UW), jnp.float32),
            pltpu.VMEM((HH, CE), jnp.float32),
            pltpu.VMEM((DK * 16,), jnp.int32),
            pltpu.VMEM((CT, 2), jnp.int32),
            pltpu.VMEM((CT,), jnp.int32),
            pltpu.VMEM((CT,), jnp.int32),
            pltpu.VMEM((CT,), jnp.int32),
            pltpu.VMEM_SHARED((N, UW), jnp.float32),
            pltpu.SemaphoreType.DMA,
            pltpu.SemaphoreType.DMA,
            pltpu.SemaphoreType.DMA,
            pltpu.SemaphoreType.DMA,
            pltpu.SemaphoreType.DMA,
            pltpu.SemaphoreType.DMA,
        ],
        compiler_params=_sc_params,
    )
    return f(k2, q2, v2, ei2, zeros)


def _ln(h, g, b, eps=1e-5):
    m = jnp.mean(h, axis=-1, keepdims=True)
    cc = h - m
    v = jnp.mean(cc * cc, axis=-1, keepdims=True)
    return cc * lax.rsqrt(v + eps) * g + b


def _post_body(x_ref, a0_ref, a1_ref, wo_ref, bo_ref, lng_ref, lnb_ref,
               w1_ref, b1_ref, w2_ref, b2_ref, ln2g_ref, ln2b_ref, out_ref):
    a0 = a0_ref[...]
    a1 = a1_ref[...]
    wv = jnp.concatenate([a0[:, :HD], a1[:, :HD]], axis=1)
    z = jnp.concatenate([a0[:, HD:HD + HH], a1[:, HD:HD + HH]], axis=1)
    m2 = (lax.broadcasted_iota(jnp.int32, (H, NDIM), 1) // DK
          == lax.broadcasted_iota(jnp.int32, (H, NDIM), 0)).astype(jnp.float32)
    zb = jnp.dot(z, m2, preferred_element_type=jnp.float32)
    o = wv / (zb + 1e-12)
    x = x_ref[...]
    h = _ln(x + jnp.dot(o, wo_ref[...], preferred_element_type=jnp.float32)
            + bo_ref[...], lng_ref[...], lnb_ref[...])
    f = jnp.maximum(jnp.dot(h, w1_ref[...], preferred_element_type=jnp.float32)
                    + b1_ref[...], 0.0)
    out_ref[...] = _ln(h + jnp.dot(f, w2_ref[...], preferred_element_type=jnp.float32)
                       + b2_ref[...], ln2g_ref[...], ln2b_ref[...])


def _post(x, a0, a1, Wo, bo, ln_g, ln_b, W1, b1, W2, b2, ln2_g, ln2_b):
    row = lambda i: (i, 0)
    fixed = lambda i: (0, 0)
    return pl.pallas_call(
        _post_body,
        grid=(N // ROW_BLK,),
        in_specs=[
            pl.BlockSpec((ROW_BLK, NDIM), row),
            pl.BlockSpec((ROW_BLK, UW), row),
            pl.BlockSpec((ROW_BLK, UW), row),
            pl.BlockSpec((NDIM, NDIM), fixed),
            pl.BlockSpec((1, NDIM), fixed),
            pl.BlockSpec((1, NDIM), fixed),
            pl.BlockSpec((1, NDIM), fixed),
            pl.BlockSpec((NDIM, DFF), fixed),
            pl.BlockSpec((1, DFF), fixed),
            pl.BlockSpec((DFF, NDIM), fixed),
            pl.BlockSpec((1, NDIM), fixed),
            pl.BlockSpec((1, NDIM), fixed),
            pl.BlockSpec((1, NDIM), fixed),
        ],
        out_specs=pl.BlockSpec((ROW_BLK, NDIM), row),
        out_shape=jax.ShapeDtypeStruct((N, NDIM), jnp.float32),
    )(x, a0, a1, Wo, bo.reshape(1, NDIM), ln_g.reshape(1, NDIM),
      ln_b.reshape(1, NDIM), W1, b1.reshape(1, DFF), W2, b2.reshape(1, NDIM),
      ln2_g.reshape(1, NDIM), ln2_b.reshape(1, NDIM))


def kernel(x, edge_index, Wq, bq, Wk, Wv, Wo, bo, ln_g, ln_b, W1, b1, W2, b2,
           ln2_g, ln2_b):
    wqkv = jnp.concatenate([Wq, Wk, Wv], axis=1)
    q, k, v = _qkv(x, wqkv, bq)

    ei2 = edge_index.T
    zeros = jnp.zeros((RPT, UW), jnp.float32)
    agg = _edge(k.reshape(2 * N, HD), q.reshape(2 * N, HD),
                v.reshape(2 * N, HD), ei2, zeros)

    return _post(x, agg[0], agg[1], Wo, bo, ln_g, ln_b, W1, b1, W2, b2,
                 ln2_g, ln2_b)


# speedup vs baseline: 1.1126x; 1.0012x over previous
import functools
import math

import jax
import jax.numpy as jnp
from jax import lax
from jax.experimental import pallas as pl
from jax.experimental.pallas import tpu as pltpu
from jax.experimental.pallas import tpu_sc as plsc

N = 10000
E = 160000
NDIM = 256
H = 8
DK = NDIM // H
DFF = 4 * NDIM

NC = 2
NS = 16
HD = NDIM // 2
HH = H // 2
UW = 144

ROW_BLK = 1000

CE = 48
NG = CE // 16
EPT = E // NS
NCH = EPT // CE
NPAIR = NCH // 2
CT = EPT - NCH * CE
RPT = N // NS
ISQ = 1.0 / math.sqrt(DK)

_mesh = plsc.VectorSubcoreMesh(
    core_axis_name="c", subcore_axis_name="s", num_cores=NC, num_subcores=NS)
_sc_params = pltpu.CompilerParams(use_tc_tiling_on_sc=False,
                                  needs_layout_passes=False)


def _qkv_body(x_ref, w_ref, bq_ref, q_ref, k_ref, v_ref):
    acc = jnp.dot(x_ref[...], w_ref[...], preferred_element_type=jnp.float32)
    q_ref[...] = acc[:, :NDIM] + bq_ref[...]
    k_ref[...] = acc[:, NDIM:2 * NDIM]
    v_ref[...] = acc[:, 2 * NDIM:]


def _qkv(x, wqkv, bq):
    out = jax.ShapeDtypeStruct((N, NDIM), jnp.float32)
    return pl.pallas_call(
        _qkv_body,
        grid=(N // ROW_BLK,),
        in_specs=[
            pl.BlockSpec((ROW_BLK, NDIM), lambda i: (i, 0)),
            pl.BlockSpec((NDIM, 3 * NDIM), lambda i: (0, 0)),
            pl.BlockSpec((1, NDIM), lambda i: (0, 0)),
        ],
        out_specs=[pl.BlockSpec((ROW_BLK, NDIM), lambda i: (i, 0))] * 3,
        out_shape=[out, out, out],
    )(x, wqkv, bq.reshape(1, NDIM))


def _edge_body(k2_hbm, q2_hbm, v2_hbm, ei2_hbm, zeros_hbm, out_hbm,
               idx2A, idx2B, sidxA, sidxB, gdidxA, gdidxB, didxA, didxB,
               didxS, kA, kB, qA, qB, vbuf, ubuf, sbufS, cvtab,
               idx2T, sidxT, gdidxT, didxT,
               accum, semA, semB, semV, semS, semIA, semIB):
    c = lax.axis_index("c")
    s = lax.axis_index("s")
    ebase = s * EPT
    rbase = s * RPT

    pltpu.sync_copy(zeros_hbm, accum.at[pl.ds(rbase, RPT)])
    pltpu.sync_copy(zeros_hbm.at[pl.ds(0, CE)], ubuf)

    def fire_idx(x, idx2, semI):
        pltpu.async_copy(ei2_hbm.at[pl.ds(ebase + x * CE, CE)], idx2, semI)

    def idx_transform(idx2, sidx, gdidx, didx, n):
        for g in range(n // 16):
            ev2 = (lax.iota(jnp.int32, 16) + g * 16) * 2
            sv = plsc.load_gather(idx2, [ev2 // 2, jnp.zeros((16,), jnp.int32)])
            dv = plsc.load_gather(idx2, [ev2 // 2, jnp.ones((16,), jnp.int32)])
            sl = pl.ds(g * 16, 16)
            sidx[sl] = sv + sv + c
            gdidx[sl] = dv + dv + c
            didx[sl] = dv

    def fire_kq(x, sidx, gdidx, kb, qb, sem):
        pltpu.async_copy(k2_hbm.at[sidx], kb, sem)
        pltpu.async_copy(q2_hbm.at[gdidx], qb, sem)

    def prep(x, idx2, semI, sidx, gdidx, didx, kb, qb, sem):
        pltpu.make_async_copy(ei2_hbm.at[pl.ds(0, CE)], idx2, semI).wait()
        idx_transform(idx2, sidx, gdidx, didx, CE)
        fire_kq(x, sidx, gdidx, kb, qb, sem)

        @pl.when(x + 2 < NCH)
        def _():
            fire_idx(x + 2, idx2, semI)

    def drain_kq(kb, qb, sem):
        pltpu.make_async_copy(k2_hbm.at[pl.ds(0, CE)], kb, sem).wait()
        pltpu.make_async_copy(k2_hbm.at[pl.ds(0, CE)], qb, sem).wait()

    _lane = lax.iota(jnp.int32, 16)

    def cvinit(ci, carry):
        cvtab[pl.ds(ci * 16, 16)] = (_lane + ci) & (DK - 1)
        return carry

    lax.fori_loop(0, DK, cvinit, 0)

    def score_loop(ngroups, kb, qb):
        def gbody(g, carry):
            ev = _lane + g * 16
            for h in range(HH):
                def cblk(b, acc):
                    cvb = cvtab[pl.ds(b * 256, 16)]
                    a0 = jnp.zeros((16,), jnp.float32)
                    a1 = jnp.zeros((16,), jnp.float32)
                    for i in range(16):
                        cv = cvb + h * DK
                        kc = plsc.load_gather(kb, [ev, cv])
                        qc = plsc.load_gather(qb, [ev, cv])
                        if i % 2 == 0:
                            a0 = a0 + kc * qc
                        else:
                            a1 = a1 + kc * qc
                        if i < 15:
                            cvb = (cvb + 1) & (DK - 1)
                    return acc + a0 + a1

                acc = lax.fori_loop(0, DK // 16, cblk,
                                    jnp.zeros((16,), jnp.float32))
                sh = jnp.exp(jnp.clip(acc * ISQ, -5.0, 5.0))
                sbufS[h, pl.ds(g * 16, 16)] = sh
            return carry

        lax.fori_loop(0, ngroups, gbody, 0)

    def weight_loop(ngroups):
        def gbody(g, carry):
            ev = _lane + g * 16
            sl = pl.ds(g * 16, 16)
            for h in range(HH):
                sv = sbufS[h, sl]
                plsc.store_scatter(
                    ubuf, [ev, jnp.full((16,), HD + h, jnp.int32)], sv)

                def cblk(b, carry2):
                    cvb = cvtab[pl.ds(b * 256, 16)]
                    for i in range(16):
                        cv = cvb + h * DK
                        vc = plsc.load_gather(vbuf, [ev, cv])
                        plsc.store_scatter(ubuf, [ev, cv], vc * sv)
                        if i < 15:
                            cvb = (cvb + 1) & (DK - 1)
                    return carry2

                lax.fori_loop(0, DK // 16, cblk, 0)
            return carry

        lax.fori_loop(0, ngroups, gbody, 0)

    def fire_scatter(didx, nrows):
        for g in range(nrows // 16):
            sl = pl.ds(g * 16, 16)
            didxS[sl] = didx[sl]
        pltpu.async_copy(ubuf.at[pl.ds(0, nrows)],
                         accum.at[didxS.at[pl.ds(0, nrows)]], semS, add=True)

    def drain_scatter(nrows):
        pltpu.make_async_copy(ubuf.at[pl.ds(0, nrows)],
                              accum.at[pl.ds(0, nrows)], semS).wait()

    plsc.subcore_barrier()

    fire_idx(0, idx2A, semIA)
    fire_idx(1, idx2B, semIB)
    prep(0, idx2A, semIA, sidxA, gdidxA, didxA, kA, qA, semA)

    def pair(j, carry):
        a = 2 * j
        prep(a + 1, idx2B, semIB, sidxB, gdidxB, didxB, kB, qB, semB)
        drain_kq(kA, qA, semA)
        pltpu.async_copy(v2_hbm.at[sidxA], vbuf, semV)
        score_loop(NG, kA, qA)
        pltpu.make_async_copy(k2_hbm.at[pl.ds(0, CE)], vbuf, semV).wait()

        @pl.when(j > 0)
        def _():
            drain_scatter(CE)

        weight_loop(NG)
        fire_scatter(didxA, CE)

        @pl.when(j < NPAIR - 1)
        def _():
            prep(a + 2, idx2A, semIA, sidxA, gdidxA, didxA, kA, qA, semA)

        drain_kq(kB, qB, semB)
        pltpu.async_copy(v2_hbm.at[sidxB], vbuf, semV)
        score_loop(NG, kB, qB)
        pltpu.make_async_copy(k2_hbm.at[pl.ds(0, CE)], vbuf, semV).wait()
        drain_scatter(CE)
        weight_loop(NG)
        fire_scatter(didxB, CE)
        return carry

    lax.fori_loop(0, NPAIR, pair, 0)
    drain_scatter(CE)

    pltpu.sync_copy(ei2_hbm.at[pl.ds(ebase + NCH * CE, CT)], idx2T)
    idx_transform(idx2T, sidxT, gdidxT, didxT, CT)
    pltpu.sync_copy(k2_hbm.at[sidxT], kA.at[pl.ds(0, CT)])
    pltpu.sync_copy(q2_hbm.at[gdidxT], qA.at[pl.ds(0, CT)])
    pltpu.sync_copy(v2_hbm.at[sidxT], vbuf.at[pl.ds(0, CT)])
    score_loop(CT // 16, kA, qA)
    weight_loop(CT // 16)
    pltpu.sync_copy(ubuf.at[pl.ds(0, CT)], accum.at[didxT], add=True)

    plsc.subcore_barrier()

    @pl.when(c == 0)
    def _():
        pltpu.sync_copy(accum.at[pl.ds(rbase, RPT)],
                        out_hbm.at[0, pl.ds(rbase, RPT)])

    @pl.when(c == 1)
    def _():
        pltpu.sync_copy(accum.at[pl.ds(rbase, RPT)],
                        out_hbm.at[1, pl.ds(rbase, RPT)])


def _edge(k2, q2, v2, ei2, zeros):
    f = pl.kernel(
        _edge_body,
        out_type=jax.ShapeDtypeStruct((2, N, UW), jnp.float32),
        mesh=_mesh,
        scratch_types=[
            pltpu.VMEM((CE, 2), jnp.int32),
            pltpu.VMEM((CE, 2), jnp.int32),
            pltpu.VMEM((CE,), jnp.int32),
            pltpu.VMEM((CE,), jnp.int32),
            pltpu.VMEM((CE,), jnp.int32),
            pltpu.VMEM((CE,), jnp.int32),
            pltpu.VMEM((CE,), jnp.int32),
            pltpu.VMEM((CE,), jnp.int32),
            pltpu.VMEM((CE,), jnp.int32),
            pltpu.VMEM((CE, HD), jnp.float32),
            pltpu.VMEM((CE, HD), jnp.float32),
            pltpu.VMEM((CE, HD), jnp.float32),
            pltpu.VMEM((CE, HD), jnp.float32),
            pltpu.VMEM((CE, HD), jnp.float32),
            pltpu.VMEM((CE, UW), jnp.float32),
            pltpu.VMEM((HH, CE), jnp.float32),
            pltpu.VMEM((DK * 16,), jnp.int32),
            pltpu.VMEM((CT, 2), jnp.int32),
            pltpu.VMEM((CT,), jnp.int32),
            pltpu.VMEM((CT,), jnp.int32),
            pltpu.VMEM((CT,), jnp.int32),
            pltpu.VMEM_SHARED((N, UW), jnp.float32),
            pltpu.SemaphoreType.DMA,
            pltpu.SemaphoreType.DMA,
            pltpu.SemaphoreType.DMA,
            pltpu.SemaphoreType.DMA,
            pltpu.SemaphoreType.DMA,
            pltpu.SemaphoreType.DMA,
        ],
        compiler_params=_sc_params,
    )
    return f(k2, q2, v2, ei2, zeros)


def _ln(h, g, b, eps=1e-5):
    m = jnp.mean(h, axis=-1, keepdims=True)
    cc = h - m
    v = jnp.mean(cc * cc, axis=-1, keepdims=True)
    return cc * lax.rsqrt(v + eps) * g + b


def _post_body(x_ref, a0_ref, a1_ref, wo_ref, bo_ref, lng_ref, lnb_ref,
               w1_ref, b1_ref, w2_ref, b2_ref, ln2g_ref, ln2b_ref, out_ref):
    a0 = a0_ref[...]
    a1 = a1_ref[...]
    wv = jnp.concatenate([a0[:, :HD], a1[:, :HD]], axis=1)
    z = jnp.concatenate([a0[:, HD:HD + HH], a1[:, HD:HD + HH]], axis=1)
    m2 = (lax.broadcasted_iota(jnp.int32, (H, NDIM), 1) // DK
          == lax.broadcasted_iota(jnp.int32, (H, NDIM), 0)).astype(jnp.float32)
    zb = jnp.dot(z, m2, preferred_element_type=jnp.float32)
    o = wv / (zb + 1e-12)
    x = x_ref[...]
    h = _ln(x + jnp.dot(o, wo_ref[...], preferred_element_type=jnp.float32)
            + bo_ref[...], lng_ref[...], lnb_ref[...])
    f = jnp.maximum(jnp.dot(h, w1_ref[...], preferred_element_type=jnp.float32)
                    + b1_ref[...], 0.0)
    out_ref[...] = _ln(h + jnp.dot(f, w2_ref[...], preferred_element_type=jnp.float32)
                       + b2_ref[...], ln2g_ref[...], ln2b_ref[...])


def _post(x, a0, a1, Wo, bo, ln_g, ln_b, W1, b1, W2, b2, ln2_g, ln2_b):
    row = lambda i: (i, 0)
    fixed = lambda i: (0, 0)
    return pl.pallas_call(
        _post_body,
        grid=(N // ROW_BLK,),
        in_specs=[
            pl.BlockSpec((ROW_BLK, NDIM), row),
            pl.BlockSpec((ROW_BLK, UW), row),
            pl.BlockSpec((ROW_BLK, UW), row),
            pl.BlockSpec((NDIM, NDIM), fixed),
            pl.BlockSpec((1, NDIM), fixed),
            pl.BlockSpec((1, NDIM), fixed),
            pl.BlockSpec((1, NDIM), fixed),
            pl.BlockSpec((NDIM, DFF), fixed),
            pl.BlockSpec((1, DFF), fixed),
            pl.BlockSpec((DFF, NDIM), fixed),
            pl.BlockSpec((1, NDIM), fixed),
            pl.BlockSpec((1, NDIM), fixed),
            pl.BlockSpec((1, NDIM), fixed),
        ],
        out_specs=pl.BlockSpec((ROW_BLK, NDIM), row),
        out_shape=jax.ShapeDtypeStruct((N, NDIM), jnp.float32),
    )(x, a0, a1, Wo, bo.reshape(1, NDIM), ln_g.reshape(1, NDIM),
      ln_b.reshape(1, NDIM), W1, b1.reshape(1, DFF), W2, b2.reshape(1, NDIM),
      ln2_g.reshape(1, NDIM), ln2_b.reshape(1, NDIM))


def kernel(x, edge_index, Wq, bq, Wk, Wv, Wo, bo, ln_g, ln_b, W1, b1, W2, b2,
           ln2_g, ln2_b):
    wqkv = jnp.concatenate([Wq, Wk, Wv], axis=1)
    q, k, v = _qkv(x, wqkv, bq)

    ei2 = edge_index.T
    zeros = jnp.zeros((RPT, UW), jnp.float32)
    agg = _edge(k.reshape(2 * N, HD), q.reshape(2 * N, HD),
                v.reshape(2 * N, HD), ei2, zeros)

    return _post(x, agg[0], agg[1], Wo, bo, ln_g, ln_b, W1, b1, W2, b2,
                 ln2_g, ln2_b)
